# Initial kernel scaffold; baseline (speedup 1.0000x reference)
import jax
import jax.numpy as jnp
from jax.experimental import pallas as pl


def kernel(node_features, edge_features, senders, receivers, W1e, b1e, W2e, b2e, W1n, b1n, W2n, b2n):
    raise NotImplementedError("write your pallas kernel here")


# speedup vs baseline: 4.4160x; 4.4160x over previous
import functools

import jax
import jax.numpy as jnp
from jax import lax
from jax.experimental import pallas as pl
from jax.experimental.pallas import tpu as pltpu
from jax.experimental.pallas import tpu_sc as plsc

N_NODES = 10000
N_EDGES = 160000
D = 256

NC = 2
NS = 16
NW = NC * NS

@functools.lru_cache(maxsize=None)
def _sc_mesh():
    return plsc.VectorSubcoreMesh(core_axis_name="c", subcore_axis_name="s",
                                  num_cores=NC, num_subcores=NS)


_G_EPW = N_EDGES // NW
_G_K = 128
_G_NCHUNK = _G_EPW // _G_K
_G_TAIL = _G_EPW - _G_NCHUNK * _G_K


def _sc_gather_body(nf_hbm, snd_hbm, rcv_hbm, out_s_hbm, out_r_hbm,
                    idx_v, rows_v, idxt_v, rowst_v, sem):
    wid = lax.axis_index("s") * NC + lax.axis_index("c")
    base = wid * _G_EPW

    def one(idx_hbm, out_hbm):
        def body(j, _):
            off = base + j * _G_K
            pltpu.sync_copy(idx_hbm.at[pl.ds(off, _G_K)], idx_v)
            pltpu.async_copy(nf_hbm.at[idx_v], rows_v, sem).wait()
            pltpu.sync_copy(rows_v, out_hbm.at[pl.ds(off, _G_K)])
            return _
        lax.fori_loop(0, _G_NCHUNK, body, 0)
        toff = base + _G_NCHUNK * _G_K
        pltpu.sync_copy(idx_hbm.at[pl.ds(toff, _G_TAIL)], idxt_v)
        pltpu.async_copy(nf_hbm.at[idxt_v], rowst_v, sem).wait()
        pltpu.sync_copy(rowst_v, out_hbm.at[pl.ds(toff, _G_TAIL)])

    one(snd_hbm, out_s_hbm)
    one(rcv_hbm, out_r_hbm)


@functools.lru_cache(maxsize=None)
def _sc_gather_kernel():
    return pl.kernel(
        _sc_gather_body,
        out_type=(jax.ShapeDtypeStruct((N_EDGES, D), jnp.float32),
                  jax.ShapeDtypeStruct((N_EDGES, D), jnp.float32)),
        mesh=_sc_mesh(),
        scratch_types=[
            pltpu.VMEM((_G_K,), jnp.int32),
            pltpu.VMEM((_G_K, D), jnp.float32),
            pltpu.VMEM((_G_TAIL,), jnp.int32),
            pltpu.VMEM((_G_TAIL, D), jnp.float32),
            pltpu.SemaphoreType.DMA,
        ],
    )


_S_H = D // NC
_S_EPW = N_EDGES // NS
_S_K = 128
_S_NCHUNK = _S_EPW // _S_K
_S_TAIL = _S_EPW - _S_NCHUNK * _S_K
_S_RPT = 624
_S_REM = N_NODES - NS * _S_RPT


def _sc_scatter_body(mij_hbm, rcv_hbm, zeros_hbm, out_hbm,
                     idx_v, rows_v, idxt_v, rowst_v, acc_sh, sem):
    c = lax.axis_index("c")
    s = lax.axis_index("s")
    col = c * _S_H
    pltpu.sync_copy(zeros_hbm.at[pl.ds(s * _S_RPT, _S_RPT)],
                    acc_sh.at[pl.ds(s * _S_RPT, _S_RPT)])
    @pl.when(s == 0)
    def _():
        pltpu.sync_copy(zeros_hbm.at[pl.ds(NS * _S_RPT, _S_REM)],
                        acc_sh.at[pl.ds(NS * _S_RPT, _S_REM)])
    plsc.subcore_barrier()

    base = s * _S_EPW

    def body(j, _):
        off = base + j * _S_K
        pltpu.sync_copy(rcv_hbm.at[pl.ds(off, _S_K)], idx_v)
        pltpu.sync_copy(mij_hbm.at[pl.ds(off, _S_K), pl.ds(col, _S_H)], rows_v)
        pltpu.sync_copy(rows_v, acc_sh.at[idx_v], add=True)
        return _
    lax.fori_loop(0, _S_NCHUNK, body, 0)
    toff = base + _S_NCHUNK * _S_K
    pltpu.sync_copy(rcv_hbm.at[pl.ds(toff, _S_TAIL)], idxt_v)
    pltpu.sync_copy(mij_hbm.at[pl.ds(toff, _S_TAIL), pl.ds(col, _S_H)], rowst_v)
    pltpu.sync_copy(rowst_v, acc_sh.at[idxt_v], add=True)

    plsc.subcore_barrier()
    pltpu.sync_copy(acc_sh.at[pl.ds(s * _S_RPT, _S_RPT)],
                    out_hbm.at[pl.ds(s * _S_RPT, _S_RPT), pl.ds(col, _S_H)])
    @pl.when(s == 0)
    def _():
        pltpu.sync_copy(acc_sh.at[pl.ds(NS * _S_RPT, _S_REM)],
                        out_hbm.at[pl.ds(NS * _S_RPT, _S_REM), pl.ds(col, _S_H)])


@functools.lru_cache(maxsize=None)
def _sc_scatter_kernel():
    return pl.kernel(
        _sc_scatter_body,
        out_type=jax.ShapeDtypeStruct((N_NODES, D), jnp.float32),
        mesh=_sc_mesh(),
        scratch_types=[
            pltpu.VMEM((_S_K,), jnp.int32),
            pltpu.VMEM((_S_K, _S_H), jnp.float32),
            pltpu.VMEM((_S_TAIL,), jnp.int32),
            pltpu.VMEM((_S_TAIL, _S_H), jnp.float32),
            pltpu.VMEM_SHARED((N_NODES, _S_H), jnp.float32),
            pltpu.SemaphoreType.DMA,
        ],
    )


_E_BLK = 1000


def _edge_mlp_body(sf_r, rf_r, ef_r, w1s_r, w1r_r, w1e_r, b1_r, w2_r, b2_r,
                   o_r):
    h = (jnp.dot(sf_r[...], w1s_r[...], preferred_element_type=jnp.float32)
         + jnp.dot(rf_r[...], w1r_r[...], preferred_element_type=jnp.float32)
         + jnp.dot(ef_r[...], w1e_r[...], preferred_element_type=jnp.float32)
         + b1_r[...])
    h = jax.nn.gelu(h)
    o_r[...] = (jnp.dot(h, w2_r[...], preferred_element_type=jnp.float32)
                + b2_r[...]) * 0.25


def _edge_mlp(sf, rf, ef, w1s, w1r, w1e, b1, w2, b2):
    nin = 3 * D
    grid = (N_EDGES // _E_BLK,)
    blk = lambda i: (i, 0)
    fixed = lambda i: (0, 0)
    return pl.pallas_call(
        _edge_mlp_body,
        grid=grid,
        in_specs=[
            pl.BlockSpec((_E_BLK, D), blk),
            pl.BlockSpec((_E_BLK, D), blk),
            pl.BlockSpec((_E_BLK, D), blk),
            pl.BlockSpec((D, nin), fixed),
            pl.BlockSpec((D, nin), fixed),
            pl.BlockSpec((D, nin), fixed),
            pl.BlockSpec((1, nin), fixed),
            pl.BlockSpec((nin, D), fixed),
            pl.BlockSpec((1, D), fixed),
        ],
        out_specs=pl.BlockSpec((_E_BLK, D), blk),
        out_shape=jax.ShapeDtypeStruct((N_EDGES, D), jnp.float32),
    )(sf, rf, ef, w1s, w1r, w1e, b1, w2, b2)


_N_BLK = 2000


def _node_mlp_body(mi_r, nf_r, w1m_r, w1x_r, b1_r, w2_r, b2_r, o_r):
    g = (jnp.dot(mi_r[...], w1m_r[...], preferred_element_type=jnp.float32)
         + jnp.dot(nf_r[...], w1x_r[...], preferred_element_type=jnp.float32)
         + b1_r[...])
    g = jax.nn.gelu(g)
    o_r[...] = (jnp.dot(g, w2_r[...], preferred_element_type=jnp.float32)
                + b2_r[...])


def _node_mlp(mi, nf, w1m, w1x, b1, w2, b2):
    nin = 2 * D
    grid = (N_NODES // _N_BLK,)
    blk = lambda i: (i, 0)
    fixed = lambda i: (0, 0)
    return pl.pallas_call(
        _node_mlp_body,
        grid=grid,
        in_specs=[
            pl.BlockSpec((_N_BLK, D), blk),
            pl.BlockSpec((_N_BLK, D), blk),
            pl.BlockSpec((D, nin), fixed),
            pl.BlockSpec((D, nin), fixed),
            pl.BlockSpec((1, nin), fixed),
            pl.BlockSpec((nin, D), fixed),
            pl.BlockSpec((1, D), fixed),
        ],
        out_specs=pl.BlockSpec((_N_BLK, D), blk),
        out_shape=jax.ShapeDtypeStruct((N_NODES, D), jnp.float32),
    )(mi, nf, w1m, w1x, b1, w2, b2)


def kernel(node_features, edge_features, senders, receivers,
           W1e, b1e, W2e, b2e, W1n, b1n, W2n, b2n):
    nf = node_features[0]
    ef = edge_features[0]
    snd = senders[0].astype(jnp.int32)
    rcv = receivers[0].astype(jnp.int32)

    sf, rf = _sc_gather_kernel()(nf, snd, rcv)

    m_ij = _edge_mlp(sf, rf, ef,
                     W1e[:D], W1e[D:2 * D], W1e[2 * D:],
                     b1e.reshape(1, -1), W2e, b2e.reshape(1, -1))

    zeros = jnp.zeros((N_NODES, _S_H), jnp.float32)
    m_i = _sc_scatter_kernel()(m_ij, rcv, zeros)

    out = _node_mlp(m_i, nf,
                    W1n[:D], W1n[D:],
                    b1n.reshape(1, -1), W2n, b2n.reshape(1, -1))
    return out[None]
